# Initial kernel scaffold; baseline (speedup 1.0000x reference)
import jax
import jax.numpy as jnp
from jax.experimental import pallas as pl


def kernel(x, edge_index, edge_weight, W_emb, b_emb, W, W_lin, W_ro, b_ro):
    raise NotImplementedError("write your pallas kernel here")


# speedup vs baseline: 2.6909x; 2.6909x over previous
import functools

import jax
import jax.numpy as jnp
from jax import lax
from jax.experimental import pallas as pl
from jax.experimental.pallas import tpu as pltpu
from jax.experimental.pallas import tpu_sc as plsc

N = 10000
E = 160000
D = 256
DH = D // 2
NUM_ITERS = 4
GAMMA = 0.1
EPS = 0.1

NTILES = 16
NCORES = 2
CH = 128
NBUF = 2
NCH = -(-(-(-E // (NTILES * CH))) // 4) * 4
NROUNDS = NCH // NBUF
EPT = NCH * CH
E_PAD = NTILES * EPT
N_PAD = 10240
RPT = N_PAD // NTILES

ROW_BLK = 1000
GRID = N // ROW_BLK

_dn = (((1,), (1,)), ((), ()))


def _mm(a, b):
    return lax.dot_general(a, b, _dn, preferred_element_type=jnp.float32)


def _tc_pro_body(x_ref, we_ref, be_ref, wl_ref, a_ref, h_ref, nb_ref, ha_ref):
    hb = _mm(x_ref[...], we_ref[...]) + be_ref[...]
    h_ref[...] = hb
    nb = _mm(hb, wl_ref[...])
    nb_ref[0] = nb[:, :DH]
    nb_ref[1] = nb[:, DH:]
    ha_ref[...] = _mm(hb, a_ref[...])


def _tc_upd_body(h_ref, ha_ref, agg_ref, wl_ref, a_ref, hn_ref, nb_ref, han_ref):
    conv = ha_ref[...] + jnp.concatenate([agg_ref[0], agg_ref[1]], axis=1)
    hn = h_ref[...] + EPS * jnp.tanh(conv)
    hn_ref[...] = hn
    nb = _mm(hn, wl_ref[...])
    nb_ref[0] = nb[:, :DH]
    nb_ref[1] = nb[:, DH:]
    han_ref[...] = _mm(hn, a_ref[...])


def _tc_ro_body(h_ref, ha_ref, agg_ref, wro_ref, bro_ref, out_ref):
    conv = ha_ref[...] + jnp.concatenate([agg_ref[0], agg_ref[1]], axis=1)
    hn = h_ref[...] + EPS * jnp.tanh(conv)
    out_ref[...] = _mm(hn, wro_ref[...]) + bro_ref[...]


_row_spec = pl.BlockSpec((ROW_BLK, D), lambda i: (i, 0))
_stk_spec = pl.BlockSpec((2, ROW_BLK, DH), lambda i: (0, i, 0))
_w_spec = pl.BlockSpec((D, D), lambda i: (0, 0))
_b_spec = pl.BlockSpec((1, D), lambda i: (0, 0))

_f32 = jnp.float32
_sds = jax.ShapeDtypeStruct

_tc_pro = pl.pallas_call(
    _tc_pro_body,
    grid=(GRID,),
    in_specs=[_row_spec, _w_spec, _b_spec, _w_spec, _w_spec],
    out_specs=[_row_spec, _stk_spec, _row_spec],
    out_shape=[_sds((N, D), _f32), _sds((2, N, DH), _f32), _sds((N, D), _f32)],
)

_tc_upd = pl.pallas_call(
    _tc_upd_body,
    grid=(GRID,),
    in_specs=[_row_spec, _row_spec, _stk_spec, _w_spec, _w_spec],
    out_specs=[_row_spec, _stk_spec, _row_spec],
    out_shape=[_sds((N, D), _f32), _sds((2, N, DH), _f32), _sds((N, D), _f32)],
)

_tc_ro = pl.pallas_call(
    _tc_ro_body,
    grid=(GRID,),
    in_specs=[_row_spec, _row_spec, _stk_spec, _w_spec, _b_spec],
    out_specs=_row_spec,
    out_shape=_sds((N, D), _f32),
)


def _sc_body(table, srcp, dstp, wp, zeros, agg_out,
             ss0, ss1, ds0, ds1, ws0, ws1,
             sc0, sc1, dc0, dc1, g0, g1,
             es0, es1, gs0, gs1, sm0, sm1, acc):
    src_st = [ss0, ss1]
    dst_st = [ds0, ds1]
    w_st = [ws0, ws1]
    src_chunk = [sc0, sc1]
    dst_chunk = [dc0, dc1]
    gath = [g0, g1]
    esem = [es0, es1]
    gsem = [gs0, gs1]
    ssem = [sm0, sm1]
    c = lax.axis_index("c")
    s = lax.axis_index("s")
    off = c * N
    out_off = c * N_PAD

    pltpu.sync_copy(zeros.at[pl.ds(s * RPT, RPT)], acc.at[pl.ds(s * RPT, RPT)])

    def stage(k, p):
        pltpu.async_copy(srcp.at[s, k], src_st[p], esem[p])
        pltpu.async_copy(dstp.at[s, k], dst_st[p], esem[p])
        pltpu.async_copy(wp.at[s, k], w_st[p], esem[p])

    def wait_stage(k, p):
        pltpu.make_async_copy(srcp.at[s, k], src_st[p], esem[p]).wait()
        pltpu.make_async_copy(dstp.at[s, k], dst_st[p], esem[p]).wait()
        pltpu.make_async_copy(wp.at[s, k], w_st[p], esem[p]).wait()

    def prep(p, b):
        for j in range(CH // 16):
            sl = pl.ds(j * 16, 16)
            src_chunk[b][sl] = src_st[p][b, sl] + off
            dst_chunk[b][sl] = dst_st[p][b, sl]

    def scale(p, b):
        def grp(g, carry2):
            wv16 = w_st[p][b, pl.ds(g * 16, 16)]
            for lane in range(16):
                wv = wv16[lane]
                e = g * 16 + lane
                for j in range(DH // 16):
                    sl = pl.ds(j * 16, 16)
                    gath[b][e, sl] = gath[b][e, sl] * wv
            return carry2

        lax.fori_loop(0, CH // 16, grp, 0)

    plsc.subcore_barrier()

    stage(0, 0)
    wait_stage(0, 0)
    for b in range(NBUF):
        prep(0, b)
        pltpu.async_copy(table.at[src_chunk[b]], gath[b], gsem[b])
    stage(1, 1)

    def pair(m, carry):
        for p in range(2):
            k = 2 * m + p
            scatters = []
            for b in range(NBUF):
                pltpu.make_async_copy(table.at[src_chunk[b]], gath[b],
                                      gsem[b]).wait()
                scale(p, b)
                scatters.append(
                    pltpu.async_copy(gath[b], acc.at[dst_chunk[b]], ssem[b],
                                     add=True))

            @pl.when(k + 2 < NROUNDS)
            def _():
                stage(k + 2, p)

            for b in range(NBUF):
                scatters[b].wait()

            @pl.when(k + 1 < NROUNDS)
            def _():
                wait_stage(k + 1, 1 - p)
                for b in range(NBUF):
                    prep(1 - p, b)
                    pltpu.async_copy(table.at[src_chunk[b]], gath[b], gsem[b])

        return carry

    lax.fori_loop(0, NROUNDS // 2, pair, 0)
    plsc.subcore_barrier()

    pltpu.sync_copy(acc.at[pl.ds(s * RPT, RPT)],
                    agg_out.at[pl.ds(out_off + s * RPT, RPT)])


_sc_agg = functools.partial(
    pl.kernel,
    out_type=_sds((2 * N_PAD, DH), _f32),
    mesh=plsc.VectorSubcoreMesh(core_axis_name="c", subcore_axis_name="s"),
    scratch_types=[
        *[pltpu.VMEM((NBUF, CH), jnp.int32) for _ in range(4)],
        *[pltpu.VMEM((NBUF, CH), _f32) for _ in range(2)],
        *[pltpu.VMEM((CH,), jnp.int32) for _ in range(2 * NBUF)],
        *[pltpu.VMEM((CH, DH), _f32) for _ in range(NBUF)],
        *[pltpu.SemaphoreType.DMA for _ in range(6)],
        pltpu.VMEM_SHARED((N_PAD, DH), _f32),
    ],
)(_sc_body)


def kernel(x, edge_index, edge_weight, W_emb, b_emb, W, W_lin, W_ro, b_ro):
    A = W - W.T - GAMMA * jnp.eye(D, dtype=W.dtype)
    src = edge_index[0].astype(jnp.int32)
    dst = edge_index[1].astype(jnp.int32)
    w = edge_weight.astype(jnp.float32)
    pad = E_PAD - E
    srcp = jnp.pad(src, (0, pad)).reshape(NTILES, NROUNDS, NBUF, CH)
    dstp = jnp.pad(dst, (0, pad)).reshape(NTILES, NROUNDS, NBUF, CH)
    wp = jnp.pad(w, (0, pad)).reshape(NTILES, NROUNDS, NBUF, CH)
    zeros = jnp.zeros((N_PAD, DH), _f32)
    be = b_emb.reshape(1, D)
    bro = b_ro.reshape(1, D)

    h, nb, ha = _tc_pro(x, W_emb, be, W_lin, A)
    for _ in range(NUM_ITERS - 1):
        agg = _sc_agg(nb.reshape(2 * N, DH), srcp, dstp, wp, zeros)
        h, nb, ha = _tc_upd(h, ha, agg.reshape(2, N_PAD, DH), W_lin, A)
    agg = _sc_agg(nb.reshape(2 * N, DH), srcp, dstp, wp, zeros)
    out = _tc_ro(h, ha, agg.reshape(2, N_PAD, DH), W_ro, bro)
    return out
